# Initial kernel scaffold; baseline (speedup 1.0000x reference)
import jax
import jax.numpy as jnp
from jax.experimental import pallas as pl


def kernel(x, edge_index, edge_type, norm, emb_rel, pm_pd, W):
    raise NotImplementedError("write your pallas kernel here")


# speedup vs baseline: 3.8813x; 3.8813x over previous
import functools

import jax
import jax.numpy as jnp
from jax import lax
from jax.experimental import pallas as pl
from jax.experimental.pallas import tpu as pltpu
from jax.experimental.pallas import tpu_sc as plsc

_NC = 2
_NS = 16
_NW = _NC * _NS
_L = 16


def _sc_aggregate(x_cols, rel_cols, src, dst, et):
    _, dp, n = x_cols.shape
    r = rel_cols.shape[2]
    e = src.shape[0]
    x_flat = x_cols.reshape(_NW, dp * n)
    rel_flat = rel_cols.reshape(_NW, dp * r)

    ch = 2000
    assert e % (2 * ch) == 0
    nch = e // ch
    gpc = ch // _L

    mesh = plsc.VectorSubcoreMesh(core_axis_name="c", subcore_axis_name="s")

    @functools.partial(
        pl.kernel,
        out_type=jax.ShapeDtypeStruct((_NW, dp * n), jnp.float32),
        mesh=mesh,
        compiler_params=pltpu.CompilerParams(needs_layout_passes=False),
        scratch_types=[
            pltpu.VMEM((dp * n,), jnp.float32),
            pltpu.VMEM((dp * r,), jnp.float32),
            pltpu.VMEM((dp * n,), jnp.float32),
            pltpu.VMEM((ch,), jnp.int32),
            pltpu.VMEM((ch,), jnp.int32),
            pltpu.VMEM((ch,), jnp.int32),
            pltpu.VMEM((ch,), jnp.int32),
            pltpu.VMEM((ch,), jnp.int32),
            pltpu.VMEM((ch,), jnp.int32),
            pltpu.SemaphoreType.DMA,
            pltpu.SemaphoreType.DMA,
            pltpu.SemaphoreType.DMA,
        ],
    )
    def agg_kernel(x_hbm, rel_hbm, src_hbm, dst_hbm, et_hbm, out_hbm,
                   xs, rs, acc, sbuf0, sbuf1, dbuf0, dbuf1, tbuf0, tbuf1,
                   sem_x, sem0, sem1):
        sbufs = (sbuf0, sbuf1)
        dbufs = (dbuf0, dbuf1)
        tbufs = (tbuf0, tbuf1)
        cid = lax.axis_index("c")
        sid = lax.axis_index("s")
        wid = sid * _NC + cid
        sems = (sem0, sem1)

        cpx = pltpu.async_copy(x_hbm.at[wid], xs, sem_x)
        cpr = pltpu.async_copy(rel_hbm.at[wid], rs, sem_x)

        def zero_body(i, carry):
            acc[pl.ds(i * _L, _L)] = jnp.zeros((_L,), jnp.float32)
            return carry
        lax.fori_loop(0, (dp * n) // _L, zero_body, 0)

        cpx.wait()
        cpr.wait()

        def start(k, slot):
            off = k * ch
            pltpu.async_copy(src_hbm.at[pl.ds(off, ch)], sbufs[slot], sems[slot])
            pltpu.async_copy(dst_hbm.at[pl.ds(off, ch)], dbufs[slot], sems[slot])
            pltpu.async_copy(et_hbm.at[pl.ds(off, ch)], tbufs[slot], sems[slot])

        def wait(k, slot):
            off = k * ch
            pltpu.make_async_copy(src_hbm.at[pl.ds(off, ch)], sbufs[slot], sems[slot]).wait()
            pltpu.make_async_copy(dst_hbm.at[pl.ds(off, ch)], dbufs[slot], sems[slot]).wait()
            pltpu.make_async_copy(et_hbm.at[pl.ds(off, ch)], tbufs[slot], sems[slot]).wait()

        start(0, 0)

        def process(slot):
            def body(g, carry):
                base = g * _L
                s = sbufs[slot][pl.ds(base, _L)]
                d = dbufs[slot][pl.ds(base, _L)]
                t = tbufs[slot][pl.ds(base, _L)]
                for c in range(dp):
                    xv = plsc.load_gather(xs, [s + jnp.int32(c * n)])
                    rv = plsc.load_gather(rs, [t + jnp.int32(c * r)])
                    plsc.addupdate_scatter(acc, [d + jnp.int32(c * n)], xv - rv)
                return carry
            lax.fori_loop(0, gpc, body, 0)

        def outer(k2, carry):
            for b in range(2):
                kk = k2 * 2 + b

                @pl.when(kk + 1 < nch)
                def _():
                    start(kk + 1, 1 - b)

                wait(kk, b)
                process(b)
            return carry
        lax.fori_loop(0, nch // 2, outer, 0)

        pltpu.sync_copy(acc, out_hbm.at[wid])

    return agg_kernel(x_flat, rel_flat, src, dst, et)


def _tc_finish(agg_n, w, norm):
    n, d = agg_n.shape
    nb = 2000
    assert n % nb == 0

    def body(a_ref, w_ref, nrm_ref, o_ref):
        o_ref[...] = jnp.dot(
            a_ref[...], w_ref[...], preferred_element_type=jnp.float32
        ) * nrm_ref[...]

    return pl.pallas_call(
        body,
        grid=(n // nb,),
        in_specs=[
            pl.BlockSpec((nb, d), lambda i: (i, 0)),
            pl.BlockSpec((d, d), lambda i: (0, 0)),
            pl.BlockSpec((nb, 1), lambda i: (i, 0)),
        ],
        out_specs=pl.BlockSpec((nb, d), lambda i: (i, 0)),
        out_shape=jax.ShapeDtypeStruct((n, d), jnp.float32),
    )(agg_n, w, norm)


def kernel(x, edge_index, edge_type, norm, emb_rel, pm_pd, W):
    n, d = x.shape
    r = emb_rel.shape[0]
    dp = d // _NW

    x_cols = x.T.reshape(_NW, dp, n)
    rel_cols = emb_rel.T.reshape(_NW, dp, r)
    src = edge_index[0].astype(jnp.int32)
    dst = edge_index[1].astype(jnp.int32)
    et = edge_type.astype(jnp.int32)

    agg = _sc_aggregate(x_cols, rel_cols, src, dst, et)
    agg_n = agg.reshape(d, n).T
    return _tc_finish(agg_n, W, norm)


# speedup vs baseline: 3.9133x; 1.0083x over previous
import functools

import jax
import jax.numpy as jnp
from jax import lax
from jax.experimental import pallas as pl
from jax.experimental.pallas import tpu as pltpu
from jax.experimental.pallas import tpu_sc as plsc

_NC = 2
_NS = 16
_NW = _NC * _NS
_L = 16
_DP = 4


def _sc_aggregate(x_cols, rel_cols, src, dst, et):
    _, dp, n = x_cols.shape
    r = rel_cols.shape[2]
    e = src.shape[0]

    ch = 2000
    assert e % (2 * ch) == 0
    nch = e // ch
    gpc = ch // _L

    mesh = plsc.VectorSubcoreMesh(core_axis_name="c", subcore_axis_name="s")

    @functools.partial(
        pl.kernel,
        out_type=jax.ShapeDtypeStruct((_NW, dp, n), jnp.float32),
        mesh=mesh,
        compiler_params=pltpu.CompilerParams(needs_layout_passes=False),
        scratch_types=(
            [pltpu.VMEM((n,), jnp.float32) for _ in range(dp)]
            + [pltpu.VMEM((r,), jnp.float32) for _ in range(dp)]
            + [pltpu.VMEM((n,), jnp.float32) for _ in range(dp)]
            + [pltpu.VMEM((ch,), jnp.int32) for _ in range(6)]
            + [pltpu.SemaphoreType.DMA] * 3
        ),
    )
    def agg_kernel(x_hbm, rel_hbm, src_hbm, dst_hbm, et_hbm, out_hbm, *refs):
        xs = refs[0:dp]
        rs = refs[dp:2 * dp]
        accs = refs[2 * dp:3 * dp]
        sbufs = refs[3 * dp:3 * dp + 2]
        dbufs = refs[3 * dp + 2:3 * dp + 4]
        tbufs = refs[3 * dp + 4:3 * dp + 6]
        sem_x, sem0, sem1 = refs[3 * dp + 6:]
        sems = (sem0, sem1)

        cid = lax.axis_index("c")
        sid = lax.axis_index("s")
        wid = sid * _NC + cid

        for c in range(dp):
            pltpu.async_copy(x_hbm.at[wid, c], xs[c], sem_x)
            pltpu.async_copy(rel_hbm.at[wid, c], rs[c], sem_x)

        def zero_body(i, carry):
            for c in range(dp):
                accs[c][pl.ds(i * _L, _L)] = jnp.zeros((_L,), jnp.float32)
            return carry
        lax.fori_loop(0, n // _L, zero_body, 0, unroll=8)

        for c in range(dp):
            pltpu.make_async_copy(x_hbm.at[wid, c], xs[c], sem_x).wait()
            pltpu.make_async_copy(rel_hbm.at[wid, c], rs[c], sem_x).wait()

        def start(k, slot):
            off = k * ch
            pltpu.async_copy(src_hbm.at[pl.ds(off, ch)], sbufs[slot], sems[slot])
            pltpu.async_copy(dst_hbm.at[pl.ds(off, ch)], dbufs[slot], sems[slot])
            pltpu.async_copy(et_hbm.at[pl.ds(off, ch)], tbufs[slot], sems[slot])

        def wait(k, slot):
            off = k * ch
            pltpu.make_async_copy(src_hbm.at[pl.ds(off, ch)], sbufs[slot], sems[slot]).wait()
            pltpu.make_async_copy(dst_hbm.at[pl.ds(off, ch)], dbufs[slot], sems[slot]).wait()
            pltpu.make_async_copy(et_hbm.at[pl.ds(off, ch)], tbufs[slot], sems[slot]).wait()

        start(0, 0)

        def process(slot):
            def body(g, carry):
                base = g * _L
                s = sbufs[slot][pl.ds(base, _L)]
                d = dbufs[slot][pl.ds(base, _L)]
                t = tbufs[slot][pl.ds(base, _L)]
                for c in range(dp):
                    xv = plsc.load_gather(xs[c], [s])
                    rv = plsc.load_gather(rs[c], [t])
                    plsc.addupdate_scatter(accs[c], [d], xv - rv)
                return carry
            lax.fori_loop(0, gpc, body, 0, unroll=4)

        def outer(k2, carry):
            for b in range(2):
                kk = k2 * 2 + b

                @pl.when(kk + 1 < nch)
                def _():
                    start(kk + 1, 1 - b)

                wait(kk, b)
                process(b)
            return carry
        lax.fori_loop(0, nch // 2, outer, 0)

        for c in range(dp):
            pltpu.sync_copy(accs[c], out_hbm.at[wid, c])

    return agg_kernel(x_cols, rel_cols, src, dst, et)


def _tc_finish(agg_n, w, norm):
    n, d = agg_n.shape
    nb = 2000
    assert n % nb == 0

    def body(a_ref, w_ref, nrm_ref, o_ref):
        o_ref[...] = jnp.dot(
            a_ref[...], w_ref[...], preferred_element_type=jnp.float32
        ) * nrm_ref[...]

    return pl.pallas_call(
        body,
        grid=(n // nb,),
        in_specs=[
            pl.BlockSpec((nb, d), lambda i: (i, 0)),
            pl.BlockSpec((d, d), lambda i: (0, 0)),
            pl.BlockSpec((nb, 1), lambda i: (i, 0)),
        ],
        out_specs=pl.BlockSpec((nb, d), lambda i: (i, 0)),
        out_shape=jax.ShapeDtypeStruct((n, d), jnp.float32),
    )(agg_n, w, norm)


def kernel(x, edge_index, edge_type, norm, emb_rel, pm_pd, W):
    n, d = x.shape
    r = emb_rel.shape[0]
    dp = d // _NW

    x_cols = x.T.reshape(_NW, dp, n)
    rel_cols = emb_rel.T.reshape(_NW, dp, r)
    src = edge_index[0].astype(jnp.int32)
    dst = edge_index[1].astype(jnp.int32)
    et = edge_type.astype(jnp.int32)

    agg = _sc_aggregate(x_cols, rel_cols, src, dst, et)
    agg_n = agg.reshape(d, n).T
    return _tc_finish(agg_n, W, norm)


# speedup vs baseline: 5.9310x; 1.5156x over previous
import functools

import jax
import jax.numpy as jnp
from jax import lax
from jax.experimental import pallas as pl
from jax.experimental.pallas import tpu as pltpu
from jax.experimental.pallas import tpu_sc as plsc

_NC = 2
_NS = 16
_NW = _NC * _NS
_L = 16
_DP = 4


def _sc_aggregate(x_cols, rel_cols, src, dst, et):
    _, dp, n = x_cols.shape
    r = rel_cols.shape[2]
    e = src.shape[0]

    ch = 2000
    assert e % (2 * ch) == 0
    nch = e // ch
    gpc = ch // _L

    mesh = plsc.VectorSubcoreMesh(core_axis_name="c", subcore_axis_name="s")

    @functools.partial(
        pl.kernel,
        out_type=jax.ShapeDtypeStruct((_NW, dp, n), jnp.float32),
        mesh=mesh,
        compiler_params=pltpu.CompilerParams(needs_layout_passes=False),
        scratch_types=(
            [pltpu.VMEM((n,), jnp.float32) for _ in range(dp)]
            + [pltpu.VMEM((r,), jnp.float32) for _ in range(dp)]
            + [pltpu.VMEM((n,), jnp.float32) for _ in range(dp)]
            + [pltpu.VMEM((ch,), jnp.int32) for _ in range(6)]
            + [pltpu.SemaphoreType.DMA] * 3
        ),
    )
    def agg_kernel(x_hbm, rel_hbm, src_hbm, dst_hbm, et_hbm, out_hbm, *refs):
        xs = refs[0:dp]
        rs = refs[dp:2 * dp]
        accs = refs[2 * dp:3 * dp]
        sbufs = refs[3 * dp:3 * dp + 2]
        dbufs = refs[3 * dp + 2:3 * dp + 4]
        tbufs = refs[3 * dp + 4:3 * dp + 6]
        sem_x, sem0, sem1 = refs[3 * dp + 6:]
        sems = (sem0, sem1)

        cid = lax.axis_index("c")
        sid = lax.axis_index("s")
        wid = sid * _NC + cid

        for c in range(dp):
            pltpu.async_copy(x_hbm.at[wid, c], xs[c], sem_x)
            pltpu.async_copy(rel_hbm.at[wid, c], rs[c], sem_x)

        def zero_body(i, carry):
            for c in range(dp):
                accs[c][pl.ds(i * _L, _L)] = jnp.zeros((_L,), jnp.float32)
            return carry
        lax.fori_loop(0, n // _L, zero_body, 0, unroll=8)

        for c in range(dp):
            pltpu.make_async_copy(x_hbm.at[wid, c], xs[c], sem_x).wait()
            pltpu.make_async_copy(rel_hbm.at[wid, c], rs[c], sem_x).wait()

        def start(k, slot):
            off = k * ch
            pltpu.async_copy(src_hbm.at[pl.ds(off, ch)], sbufs[slot], sems[slot])
            pltpu.async_copy(dst_hbm.at[pl.ds(off, ch)], dbufs[slot], sems[slot])
            pltpu.async_copy(et_hbm.at[pl.ds(off, ch)], tbufs[slot], sems[slot])

        def wait(k, slot):
            off = k * ch
            pltpu.make_async_copy(src_hbm.at[pl.ds(off, ch)], sbufs[slot], sems[slot]).wait()
            pltpu.make_async_copy(dst_hbm.at[pl.ds(off, ch)], dbufs[slot], sems[slot]).wait()
            pltpu.make_async_copy(et_hbm.at[pl.ds(off, ch)], tbufs[slot], sems[slot]).wait()

        start(0, 0)

        def process(slot):
            def body(g, carry):
                base = g * _L
                s = sbufs[slot][pl.ds(base, _L)]
                t = tbufs[slot][pl.ds(base, _L)]
                d = dbufs[slot][pl.ds(base, _L)]
                xvs = [plsc.load_gather(xs[c], [s]) for c in range(dp)]
                rvs = [plsc.load_gather(rs[c], [t]) for c in range(dp)]
                for c in range(dp):
                    plsc.addupdate_scatter(accs[c], [d], xvs[c] - rvs[c])
                return carry
            lax.fori_loop(0, gpc, body, 0, unroll=4)

        def outer(k2, carry):
            for b in range(2):
                kk = k2 * 2 + b

                @pl.when(kk + 1 < nch)
                def _():
                    start(kk + 1, 1 - b)

                wait(kk, b)
                process(b)
            return carry
        lax.fori_loop(0, nch // 2, outer, 0)

        for c in range(dp):
            pltpu.sync_copy(accs[c], out_hbm.at[wid, c])

    return agg_kernel(x_cols, rel_cols, src, dst, et)


def _tc_finish(agg_n, w, norm):
    n, d = agg_n.shape
    nb = 2000
    assert n % nb == 0

    def body(a_ref, w_ref, nrm_ref, o_ref):
        o_ref[...] = jnp.dot(
            a_ref[...], w_ref[...], preferred_element_type=jnp.float32
        ) * nrm_ref[...]

    return pl.pallas_call(
        body,
        grid=(n // nb,),
        in_specs=[
            pl.BlockSpec((nb, d), lambda i: (i, 0)),
            pl.BlockSpec((d, d), lambda i: (0, 0)),
            pl.BlockSpec((nb, 1), lambda i: (i, 0)),
        ],
        out_specs=pl.BlockSpec((nb, d), lambda i: (i, 0)),
        out_shape=jax.ShapeDtypeStruct((n, d), jnp.float32),
    )(agg_n, w, norm)


def kernel(x, edge_index, edge_type, norm, emb_rel, pm_pd, W):
    n, d = x.shape
    r = emb_rel.shape[0]
    dp = d // _NW

    x_cols = x.T.reshape(_NW, dp, n)
    rel_cols = emb_rel.T.reshape(_NW, dp, r)
    src = edge_index[0].astype(jnp.int32)
    dst = edge_index[1].astype(jnp.int32)
    et = edge_type.astype(jnp.int32)

    agg = _sc_aggregate(x_cols, rel_cols, src, dst, et)
    agg_n = agg.reshape(d, n).T
    return _tc_finish(agg_n, W, norm)


# speedup vs baseline: 7.4307x; 1.2529x over previous
import functools

import jax
import jax.numpy as jnp
from jax import lax
from jax.experimental import pallas as pl
from jax.experimental.pallas import tpu as pltpu
from jax.experimental.pallas import tpu_sc as plsc

_NC = 2
_NS = 16
_NW = _NC * _NS
_L = 16
_DP = 4


def _sc_aggregate(x_cols, rel_cols, src, dst, et):
    _, dp, n = x_cols.shape
    r = rel_cols.shape[2]
    e = src.shape[0]

    ch = 2000
    assert e % (2 * ch) == 0
    nch = e // ch
    gpc = ch // _L

    mesh = plsc.VectorSubcoreMesh(core_axis_name="c", subcore_axis_name="s")

    @functools.partial(
        pl.kernel,
        out_type=jax.ShapeDtypeStruct((_NW, dp, n), jnp.float32),
        mesh=mesh,
        compiler_params=pltpu.CompilerParams(needs_layout_passes=False),
        scratch_types=(
            [pltpu.VMEM((n,), jnp.float32) for _ in range(dp)]
            + [pltpu.VMEM((r,), jnp.float32) for _ in range(dp)]
            + [pltpu.VMEM((n,), jnp.float32) for _ in range(dp)]
            + [pltpu.VMEM((ch,), jnp.int32) for _ in range(6)]
            + [pltpu.SemaphoreType.DMA] * 3
        ),
    )
    def agg_kernel(x_hbm, rel_hbm, src_hbm, dst_hbm, et_hbm, out_hbm, *refs):
        xs = refs[0:dp]
        rs = refs[dp:2 * dp]
        accs = refs[2 * dp:3 * dp]
        sbufs = refs[3 * dp:3 * dp + 2]
        dbufs = refs[3 * dp + 2:3 * dp + 4]
        tbufs = refs[3 * dp + 4:3 * dp + 6]
        sem_x, sem0, sem1 = refs[3 * dp + 6:]
        sems = (sem0, sem1)

        cid = lax.axis_index("c")
        sid = lax.axis_index("s")
        wid = sid * _NC + cid

        for c in range(dp):
            pltpu.async_copy(x_hbm.at[wid, c], xs[c], sem_x)
            pltpu.async_copy(rel_hbm.at[wid, c], rs[c], sem_x)

        def zero_body(i, carry):
            for c in range(dp):
                accs[c][pl.ds(i * _L, _L)] = jnp.zeros((_L,), jnp.float32)
            return carry
        lax.fori_loop(0, n // _L, zero_body, 0, unroll=8)

        for c in range(dp):
            pltpu.make_async_copy(x_hbm.at[wid, c], xs[c], sem_x).wait()
            pltpu.make_async_copy(rel_hbm.at[wid, c], rs[c], sem_x).wait()

        def start(k, slot):
            off = k * ch
            pltpu.async_copy(src_hbm.at[pl.ds(off, ch)], sbufs[slot], sems[slot])
            pltpu.async_copy(dst_hbm.at[pl.ds(off, ch)], dbufs[slot], sems[slot])
            pltpu.async_copy(et_hbm.at[pl.ds(off, ch)], tbufs[slot], sems[slot])

        def wait(k, slot):
            off = k * ch
            pltpu.make_async_copy(src_hbm.at[pl.ds(off, ch)], sbufs[slot], sems[slot]).wait()
            pltpu.make_async_copy(dst_hbm.at[pl.ds(off, ch)], dbufs[slot], sems[slot]).wait()
            pltpu.make_async_copy(et_hbm.at[pl.ds(off, ch)], tbufs[slot], sems[slot]).wait()

        start(0, 0)

        def process(slot):
            sb, tb, db = sbufs[slot], tbufs[slot], dbufs[slot]

            def compute(s, t, d):
                xvs = [plsc.load_gather(xs[c], [s]) for c in range(dp)]
                rvs = [plsc.load_gather(rs[c], [t]) for c in range(dp)]
                for c in range(dp):
                    plsc.addupdate_scatter(accs[c], [d], xvs[c] - rvs[c])

            def body(g, carry):
                s, t, d = carry
                base = (g + 1) * _L
                s2 = sb[pl.ds(base, _L)]
                t2 = tb[pl.ds(base, _L)]
                d2 = db[pl.ds(base, _L)]
                compute(s, t, d)
                return (s2, t2, d2)

            first = (sb[pl.ds(0, _L)], tb[pl.ds(0, _L)], db[pl.ds(0, _L)])
            last = lax.fori_loop(0, gpc - 1, body, first, unroll=4)
            compute(*last)

        def outer(k2, carry):
            for b in range(2):
                kk = k2 * 2 + b

                @pl.when(kk + 1 < nch)
                def _():
                    start(kk + 1, 1 - b)

                wait(kk, b)
                process(b)
            return carry
        lax.fori_loop(0, nch // 2, outer, 0)

        for c in range(dp):
            pltpu.sync_copy(accs[c], out_hbm.at[wid, c])

    return agg_kernel(x_cols, rel_cols, src, dst, et)


def _tc_finish(agg_n, w, norm):
    n, d = agg_n.shape
    nb = 2000
    assert n % nb == 0

    def body(a_ref, w_ref, nrm_ref, o_ref):
        o_ref[...] = jnp.dot(
            a_ref[...], w_ref[...], preferred_element_type=jnp.float32
        ) * nrm_ref[...]

    return pl.pallas_call(
        body,
        grid=(n // nb,),
        in_specs=[
            pl.BlockSpec((nb, d), lambda i: (i, 0)),
            pl.BlockSpec((d, d), lambda i: (0, 0)),
            pl.BlockSpec((nb, 1), lambda i: (i, 0)),
        ],
        out_specs=pl.BlockSpec((nb, d), lambda i: (i, 0)),
        out_shape=jax.ShapeDtypeStruct((n, d), jnp.float32),
    )(agg_n, w, norm)


def kernel(x, edge_index, edge_type, norm, emb_rel, pm_pd, W):
    n, d = x.shape
    r = emb_rel.shape[0]
    dp = d // _NW

    x_cols = x.T.reshape(_NW, dp, n)
    rel_cols = emb_rel.T.reshape(_NW, dp, r)
    src = edge_index[0].astype(jnp.int32)
    dst = edge_index[1].astype(jnp.int32)
    et = edge_type.astype(jnp.int32)

    agg = _sc_aggregate(x_cols, rel_cols, src, dst, et)
    agg_n = agg.reshape(d, n).T
    return _tc_finish(agg_n, W, norm)
